# baseline (speedup 1.0000x reference)
import functools

import jax
import jax.numpy as jnp
from jax import lax
from jax.experimental import pallas as pl
from jax.experimental.pallas import tpu as pltpu
from jax.experimental.pallas import tpu_sc as plsc

_NQ = 4
_K = 1024
_D = 64
_N = 18432
_NW = 32
_BPW = _N // _NW


def _tc_level_kernel(first, r_ref, q_ref, cb_ref, ex_ref, cn_ref,
                     idx_ref, cnt_ref, rout_ref, loss_ref):
    j = pl.program_id(0)

    @pl.when(j == 0)
    def _init():
        cnt_ref[...] = jnp.zeros_like(cnt_ref)
        loss_ref[...] = jnp.zeros_like(loss_ref)

    r = r_ref[...]
    b = r.shape[0]
    if not first:
        q = q_ref[:, :_D]
        q_st = r + (q - r)
        diff = r - q
        e = diff * diff
        loss_ref[...] += jnp.sum(jnp.mean(e + 0.25 * e, axis=1)).reshape(1, 1)
        r = r - q_st
    rout_ref[...] = r

    col_iota = jax.lax.broadcasted_iota(jnp.int32, (b, _K), 1)
    ones_row = jnp.ones((1, b), dtype=jnp.bfloat16)
    cn = cn_ref[...]
    s2 = jax.lax.dot_general(-2.0 * r, cb_ref[...], (((1,), (1,)), ((), ())),
                             preferred_element_type=jnp.float32)
    rn = jnp.sum(r * r, axis=1, keepdims=True)
    d2 = (rn + s2) + cn
    m = jnp.min(d2, axis=1, keepdims=True)
    oh0 = d2 == m
    ohb = oh0.astype(jnp.bfloat16)

    def _cols(maskb):
        p = jax.lax.dot_general(maskb, ex_ref[...], (((1,), (0,)), ((), ())),
                                preferred_element_type=jnp.float32)
        idxv = (p[:, 0:1] + p[:, 1:2]).astype(jnp.int32)
        cntv = jax.lax.dot_general(ones_row, maskb, (((1,), (0,)), ((), ())),
                                   preferred_element_type=jnp.float32)
        return idxv, p[:, 2:3], cntv

    idx, mult, cnt = _cols(ohb)

    def _tie_fix(_):
        i1 = jnp.min(jnp.where(oh0, col_iota, _K), axis=1, keepdims=True)
        _, _, c1 = _cols((col_iota == i1).astype(jnp.bfloat16))
        return i1, c1

    idx, cnt = jax.lax.cond(jnp.max(mult) > 1.5, _tie_fix,
                            lambda _: (idx, cnt), None)
    idx_ref[...] = idx
    cnt_ref[...] += cnt


def _tc_final_kernel(x_ref, r_ref, q_ref, quant_ref, loss_ref):
    j = pl.program_id(0)

    @pl.when(j == 0)
    def _init():
        loss_ref[...] = jnp.zeros_like(loss_ref)

    r = r_ref[...]
    q = q_ref[:, :_D]
    q_st = r + (q - r)
    diff = r - q
    e = diff * diff
    loss_ref[...] += jnp.sum(jnp.mean(e + 0.25 * e, axis=1)).reshape(1, 1)
    quant_ref[...] = x_ref[...] - (r - q_st)


_sc_mesh = plsc.VectorSubcoreMesh(core_axis_name="c", subcore_axis_name="s")


@functools.partial(
    pl.kernel,
    out_type=jax.ShapeDtypeStruct((_N, 2 * _D), jnp.float32),
    mesh=_sc_mesh,
    scratch_types=[
        pltpu.VMEM((_BPW,), jnp.int32),
        pltpu.VMEM((_BPW, 2 * _D), jnp.float32),
        pltpu.SemaphoreType.DMA,
    ],
)
def _sc_gather(table_hbm, idx_hbm, out_hbm, idx_v, rows_v, sem):
    wid = lax.axis_index("s") * 2 + lax.axis_index("c")
    base = wid * _BPW
    pltpu.sync_copy(idx_hbm.at[pl.ds(base, _BPW)], idx_v)
    pltpu.async_copy(table_hbm.at[idx_v], rows_v, sem).wait()
    pltpu.sync_copy(rows_v, out_hbm.at[pl.ds(base, _BPW)])


def _tc_level(r, qin, cb_i, ex, cn_i, first, blk):
    grid = (_N // blk,)
    return pl.pallas_call(
        functools.partial(_tc_level_kernel, first),
        grid=grid,
        in_specs=[
            pl.BlockSpec((blk, _D), lambda j: (j, 0)),
            pl.BlockSpec((blk, 2 * _D), lambda j: (j, 0)),
            pl.BlockSpec((_K, _D), lambda j: (0, 0)),
            pl.BlockSpec((_K, 3), lambda j: (0, 0)),
            pl.BlockSpec((1, _K), lambda j: (0, 0)),
        ],
        out_specs=[
            pl.BlockSpec((blk, 1), lambda j: (j, 0)),
            pl.BlockSpec((1, _K), lambda j: (0, 0)),
            pl.BlockSpec((blk, _D), lambda j: (j, 0)),
            pl.BlockSpec((1, 1), lambda j: (0, 0)),
        ],
        out_shape=[
            jax.ShapeDtypeStruct((_N, 1), jnp.int32),
            jax.ShapeDtypeStruct((1, _K), jnp.float32),
            jax.ShapeDtypeStruct((_N, _D), jnp.float32),
            jax.ShapeDtypeStruct((1, 1), jnp.float32),
        ],
        compiler_params=pltpu.CompilerParams(
            dimension_semantics=("arbitrary",)),
    )(r, qin, cb_i, ex, cn_i)


def kernel(inputs, codebooks):
    shape = inputs.shape
    d = shape[-1]
    flat = inputs.reshape(-1, d)
    n = flat.shape[0]
    nq, k, _ = codebooks.shape
    cnorm = jnp.stack(
        [jnp.sum(codebooks[i] * codebooks[i], axis=1) for i in range(nq)],
        axis=0)
    iota = jnp.arange(k, dtype=jnp.int32)
    ex = jnp.stack([(iota & ~3).astype(jnp.float32),
                    (iota & 3).astype(jnp.float32),
                    jnp.ones((k,), jnp.float32)], axis=1).astype(jnp.bfloat16)

    blk = 1024
    cb_pad = jnp.concatenate(
        [codebooks, jnp.zeros((nq, k, d), codebooks.dtype)], axis=2)
    r = flat
    q_prev = jnp.zeros((n, 2 * d), jnp.float32)
    idx_list, cnt_list, loss_parts = [], [], []
    for i in range(nq):
        idx_i, cnt_i, r, loss_i = _tc_level(
            r, q_prev, codebooks[i], ex, cnorm[i:i + 1, :], i == 0, blk)
        q_prev = _sc_gather(cb_pad[i], idx_i.reshape(n))
        idx_list.append(idx_i)
        cnt_list.append(cnt_i)
        loss_parts.append(loss_i)

    quant, loss_f = pl.pallas_call(
        _tc_final_kernel,
        grid=(n // blk,),
        in_specs=[
            pl.BlockSpec((blk, d), lambda j: (j, 0)),
            pl.BlockSpec((blk, d), lambda j: (j, 0)),
            pl.BlockSpec((blk, 2 * d), lambda j: (j, 0)),
        ],
        out_specs=[
            pl.BlockSpec((blk, d), lambda j: (j, 0)),
            pl.BlockSpec((1, 1), lambda j: (0, 0)),
        ],
        out_shape=[
            jax.ShapeDtypeStruct((n, d), jnp.float32),
            jax.ShapeDtypeStruct((1, 1), jnp.float32),
        ],
        compiler_params=pltpu.CompilerParams(
            dimension_semantics=("arbitrary",)),
    )(flat, r, q_prev)

    qloss = (loss_parts[1][0, 0] + loss_parts[2][0, 0]
             + loss_parts[3][0, 0] + loss_f[0, 0]) / jnp.float32(n)
    quantized = quant.reshape(shape)
    qloss_out = jnp.full(shape[:-1] + (1,), qloss, dtype=jnp.float32)
    nn_idx = jnp.concatenate(idx_list, axis=1).T.reshape((nq,) + shape[:-1])
    codebooks_out = codebooks.reshape(-1, d)
    counts = jnp.concatenate(cnt_list, axis=0).astype(jnp.int32)
    return quantized, qloss_out, nn_idx, codebooks_out, counts


# speedup vs baseline: 1.5243x; 1.5243x over previous
import jax
import jax.numpy as jnp
from jax.experimental import pallas as pl
from jax.experimental.pallas import tpu as pltpu

_NQ = 4
_K = 1024
_D = 64


def _rvq_block_kernel(x_ref, cb_ref, cb3_ref, cn_ref, quant_ref, nn_ref,
                      counts_ref, loss_ref):
    j = pl.program_id(0)

    @pl.when(j == 0)
    def _init():
        counts_ref[...] = jnp.zeros_like(counts_ref)
        loss_ref[...] = jnp.zeros_like(loss_ref)

    nh = 2
    b2 = x_ref.shape[0]
    b = b2 // nh
    col_iota = jax.lax.broadcasted_iota(jnp.int32, (b, _K), 1)
    ones_row = jnp.ones((1, b), dtype=jnp.bfloat16)

    def _from_mask(maskb, i):
        p = jax.lax.dot_general(maskb, cb3_ref[i],
                                (((1,), (0,)), ((), ())),
                                preferred_element_type=jnp.float32)
        qv = (p[:, :_D] + p[:, _D:2 * _D]) + p[:, 2 * _D:3 * _D]
        idxv = (p[:, 3 * _D:3 * _D + 1]
                + p[:, 3 * _D + 1:3 * _D + 2]).astype(jnp.int32)
        multv = p[:, 3 * _D + 2:3 * _D + 3]
        cntv = jax.lax.dot_general(ones_row, maskb,
                                   (((1,), (0,)), ((), ())),
                                   preferred_element_type=jnp.float32)
        return qv, idxv, multv, cntv

    def _level(r, i):
        cb = cb_ref[i]
        cn = cn_ref[i:i + 1, :]
        s2 = jax.lax.dot_general(-2.0 * r, cb, (((1,), (1,)), ((), ())),
                                 preferred_element_type=jnp.float32)
        rn = jnp.sum(r * r, axis=1, keepdims=True)
        d2 = (rn + s2) + cn
        m = jnp.min(d2, axis=1, keepdims=True)
        oh0 = d2 == m
        q, idx, mult, cnt = _from_mask(oh0.astype(jnp.bfloat16), i)
        return q, idx, mult, cnt, oh0

    rs = [x_ref[h * b:(h + 1) * b, :] for h in range(nh)]
    qsums = [jnp.zeros_like(rs[h]) for h in range(nh)]
    loss_sum = jnp.float32(0.0)
    nn_cols = [[] for _ in range(nh)]
    cnt_rows = []
    for i in range(_NQ):
        lv = [_level(rs[h], i) for h in range(nh)]
        qs = [t[0] for t in lv]
        idxs = [t[1] for t in lv]
        cnts = [t[3] for t in lv]

        def _tie_fix(_):
            out = []
            for h in range(nh):
                ih = jnp.min(jnp.where(lv[h][4], col_iota, _K), axis=1,
                             keepdims=True)
                qh, _, _, ch = _from_mask(
                    (col_iota == ih).astype(jnp.bfloat16), i)
                out.extend([qh, ih, ch])
            return tuple(out)

        any_tie = lv[0][2]
        for h in range(1, nh):
            any_tie = jnp.maximum(any_tie, lv[h][2])
        flat_fix = jax.lax.cond(
            jnp.max(any_tie) > 1.5, _tie_fix,
            lambda _: tuple(v for h in range(nh)
                            for v in (qs[h], idxs[h], cnts[h])), None)
        qs = [flat_fix[3 * h] for h in range(nh)]
        idxs = [flat_fix[3 * h + 1] for h in range(nh)]
        cnts = [flat_fix[3 * h + 2] for h in range(nh)]

        cnt_lv = cnts[0]
        for h in range(nh):
            q_st = rs[h] + (qs[h] - rs[h])
            qsums[h] = qsums[h] + q_st
            dh = rs[h] - qs[h]
            eh = dh * dh
            loss_sum = loss_sum + jnp.sum(jnp.mean(eh + 0.25 * eh, axis=1))
            nn_cols[h].append(idxs[h])
            rs[h] = rs[h] - q_st
            if h > 0:
                cnt_lv = cnt_lv + cnts[h]
        cnt_rows.append(cnt_lv)
    for h in range(nh):
        quant_ref[h * b:(h + 1) * b, :] = qsums[h]
        nn_ref[h * b:(h + 1) * b, :] = jnp.concatenate(nn_cols[h], axis=1)
    counts_ref[...] += jnp.concatenate(cnt_rows, axis=0)
    loss_ref[...] += loss_sum.reshape(1, 1)


def kernel(inputs, codebooks):
    shape = inputs.shape
    d = shape[-1]
    flat = inputs.reshape(-1, d)
    n = flat.shape[0]
    nq, k, _ = codebooks.shape
    cnorm = jnp.stack(
        [jnp.sum(codebooks[i] * codebooks[i], axis=1) for i in range(nq)],
        axis=0)
    mask = jnp.uint32(0xFFFF0000)
    u = codebooks
    hi = jax.lax.bitcast_convert_type(
        jax.lax.bitcast_convert_type(u, jnp.uint32) & mask, jnp.float32)
    r1 = u - hi
    mid = jax.lax.bitcast_convert_type(
        jax.lax.bitcast_convert_type(r1, jnp.uint32) & mask, jnp.float32)
    lo = r1 - mid
    iota = jnp.arange(k, dtype=jnp.int32)
    extra = jnp.stack([(iota & ~3).astype(jnp.float32),
                       (iota & 3).astype(jnp.float32),
                       jnp.ones((k,), jnp.float32)], axis=1)
    cb3 = jnp.concatenate(
        [hi, mid, lo, jnp.broadcast_to(extra[None], (nq, k, 3))],
        axis=-1).astype(jnp.bfloat16)
    blk = 768
    grid = (n // blk,)
    quant, nn, counts, loss = pl.pallas_call(
        _rvq_block_kernel,
        grid=grid,
        in_specs=[
            pl.BlockSpec((blk, d), lambda j: (j, 0)),
            pl.BlockSpec((nq, k, d), lambda j: (0, 0, 0)),
            pl.BlockSpec((nq, k, 3 * d + 3), lambda j: (0, 0, 0)),
            pl.BlockSpec((nq, k), lambda j: (0, 0)),
        ],
        out_specs=[
            pl.BlockSpec((blk, d), lambda j: (j, 0)),
            pl.BlockSpec((blk, nq), lambda j: (j, 0)),
            pl.BlockSpec((nq, k), lambda j: (0, 0)),
            pl.BlockSpec((1, 1), lambda j: (0, 0)),
        ],
        out_shape=[
            jax.ShapeDtypeStruct((n, d), jnp.float32),
            jax.ShapeDtypeStruct((n, nq), jnp.int32),
            jax.ShapeDtypeStruct((nq, k), jnp.float32),
            jax.ShapeDtypeStruct((1, 1), jnp.float32),
        ],
        compiler_params=pltpu.CompilerParams(
            dimension_semantics=("arbitrary",)),
    )(flat, codebooks, cb3, cnorm)
    quantized = quant.reshape(shape)
    qloss = loss[0, 0] / jnp.float32(n)
    qloss_out = jnp.full(shape[:-1] + (1,), qloss, dtype=jnp.float32)
    nn_idx = nn.T.reshape((nq,) + shape[:-1])
    codebooks_out = codebooks.reshape(-1, d)
    return quantized, qloss_out, nn_idx, codebooks_out, counts.astype(jnp.int32)


# speedup vs baseline: 1.6240x; 1.0654x over previous
import jax
import jax.numpy as jnp
from jax.experimental import pallas as pl
from jax.experimental.pallas import tpu as pltpu

_NQ = 4
_K = 1024
_D = 64


def _rvq_block_kernel(x_ref, cb_ref, cb3_ref, cn_ref, quant_ref, nn_ref,
                      counts_ref, loss_ref):
    j = pl.program_id(0)

    @pl.when(j == 0)
    def _init():
        counts_ref[...] = jnp.zeros_like(counts_ref)
        loss_ref[...] = jnp.zeros_like(loss_ref)

    nh = 2
    b2 = x_ref.shape[0]
    b = b2 // nh
    col_iota = jax.lax.broadcasted_iota(jnp.int32, (b, _K), 1)
    ones_row = jnp.ones((1, b), dtype=jnp.bfloat16)

    def _from_mask(maskb, i):
        p = jax.lax.dot_general(maskb, cb3_ref[i],
                                (((1,), (0,)), ((), ())),
                                preferred_element_type=jnp.float32)
        qv = (p[:, :_D] + p[:, _D:2 * _D]) + p[:, 2 * _D:3 * _D]
        idxv = (p[:, 3 * _D:3 * _D + 1]
                + p[:, 3 * _D + 1:3 * _D + 2]).astype(jnp.int32)
        multv = p[:, 3 * _D + 2:3 * _D + 3]
        cntv = jax.lax.dot_general(ones_row, maskb,
                                   (((1,), (0,)), ((), ())),
                                   preferred_element_type=jnp.float32)
        return qv, idxv, multv, cntv

    def _level(r, i):
        cb = cb_ref[i]
        cn = cn_ref[i:i + 1, :]
        s2 = jax.lax.dot_general(-2.0 * r, cb, (((1,), (1,)), ((), ())),
                                 preferred_element_type=jnp.float32)
        rn = jnp.sum(r * r, axis=1, keepdims=True)
        d2 = (rn + s2) + cn
        m = jnp.min(d2, axis=1, keepdims=True)
        oh0 = d2 == m
        q, idx, mult, cnt = _from_mask(oh0.astype(jnp.bfloat16), i)
        return q, idx, mult, cnt, oh0

    rs = [x_ref[h * b:(h + 1) * b, :] for h in range(nh)]
    qsums = [jnp.zeros_like(rs[h]) for h in range(nh)]
    loss_sum = jnp.float32(0.0)
    nn_cols = [[] for _ in range(nh)]
    cnt_rows = []
    for i in range(_NQ):
        lv = [_level(rs[h], i) for h in range(nh)]
        qs = [t[0] for t in lv]
        idxs = [t[1] for t in lv]
        cnts = [t[3] for t in lv]

        def _tie_fix(_):
            out = []
            for h in range(nh):
                ih = jnp.min(jnp.where(lv[h][4], col_iota, _K), axis=1,
                             keepdims=True)
                qh, _, _, ch = _from_mask(
                    (col_iota == ih).astype(jnp.bfloat16), i)
                out.extend([qh, ih, ch])
            return tuple(out)

        any_tie = lv[0][2]
        for h in range(1, nh):
            any_tie = jnp.maximum(any_tie, lv[h][2])
        flat_fix = jax.lax.cond(
            jnp.max(any_tie) > 1.5, _tie_fix,
            lambda _: tuple(v for h in range(nh)
                            for v in (qs[h], idxs[h], cnts[h])), None)
        qs = [flat_fix[3 * h] for h in range(nh)]
        idxs = [flat_fix[3 * h + 1] for h in range(nh)]
        cnts = [flat_fix[3 * h + 2] for h in range(nh)]

        cnt_lv = cnts[0]
        for h in range(nh):
            q_st = rs[h] + (qs[h] - rs[h])
            qsums[h] = qsums[h] + q_st
            dh = rs[h] - qs[h]
            eh = dh * dh
            loss_sum = loss_sum + jnp.sum(jnp.mean(eh + 0.25 * eh, axis=1))
            nn_cols[h].append(idxs[h])
            rs[h] = rs[h] - q_st
            if h > 0:
                cnt_lv = cnt_lv + cnts[h]
        cnt_rows.append(cnt_lv)
    for h in range(nh):
        quant_ref[h * b:(h + 1) * b, :] = qsums[h]
        nn_ref[h * b:(h + 1) * b, :] = jnp.concatenate(nn_cols[h], axis=1)
    counts_ref[...] += jnp.concatenate(cnt_rows, axis=0)
    loss_ref[...] += loss_sum.reshape(1, 1)


def kernel(inputs, codebooks):
    shape = inputs.shape
    d = shape[-1]
    flat = inputs.reshape(-1, d)
    n = flat.shape[0]
    nq, k, _ = codebooks.shape
    cnorm = jnp.stack(
        [jnp.sum(codebooks[i] * codebooks[i], axis=1) for i in range(nq)],
        axis=0)
    mask = jnp.uint32(0xFFFF0000)
    u = codebooks
    hi = jax.lax.bitcast_convert_type(
        jax.lax.bitcast_convert_type(u, jnp.uint32) & mask, jnp.float32)
    r1 = u - hi
    mid = jax.lax.bitcast_convert_type(
        jax.lax.bitcast_convert_type(r1, jnp.uint32) & mask, jnp.float32)
    lo = r1 - mid
    iota = jnp.arange(k, dtype=jnp.int32)
    extra = jnp.stack([(iota & ~3).astype(jnp.float32),
                       (iota & 3).astype(jnp.float32),
                       jnp.ones((k,), jnp.float32)], axis=1)
    cb3 = jnp.concatenate(
        [hi, mid, lo, jnp.broadcast_to(extra[None], (nq, k, 3))],
        axis=-1).astype(jnp.bfloat16)
    blk = 1024
    grid = (n // blk,)
    quant, nn, counts, loss = pl.pallas_call(
        _rvq_block_kernel,
        grid=grid,
        in_specs=[
            pl.BlockSpec((blk, d), lambda j: (j, 0)),
            pl.BlockSpec((nq, k, d), lambda j: (0, 0, 0)),
            pl.BlockSpec((nq, k, 3 * d + 3), lambda j: (0, 0, 0)),
            pl.BlockSpec((nq, k), lambda j: (0, 0)),
        ],
        out_specs=[
            pl.BlockSpec((blk, d), lambda j: (j, 0)),
            pl.BlockSpec((blk, nq), lambda j: (j, 0)),
            pl.BlockSpec((nq, k), lambda j: (0, 0)),
            pl.BlockSpec((1, 1), lambda j: (0, 0)),
        ],
        out_shape=[
            jax.ShapeDtypeStruct((n, d), jnp.float32),
            jax.ShapeDtypeStruct((n, nq), jnp.int32),
            jax.ShapeDtypeStruct((nq, k), jnp.float32),
            jax.ShapeDtypeStruct((1, 1), jnp.float32),
        ],
        compiler_params=pltpu.CompilerParams(
            dimension_semantics=("arbitrary",)),
    )(flat, codebooks, cb3, cnorm)
    quantized = quant.reshape(shape)
    qloss = loss[0, 0] / jnp.float32(n)
    qloss_out = jnp.full(shape[:-1] + (1,), qloss, dtype=jnp.float32)
    nn_idx = nn.T.reshape((nq,) + shape[:-1])
    codebooks_out = codebooks.reshape(-1, d)
    return quantized, qloss_out, nn_idx, codebooks_out, counts.astype(jnp.int32)
